# Initial kernel scaffold; baseline (speedup 1.0000x reference)
import functools

import jax
import jax.numpy as jnp
from jax import lax
from jax.experimental import pallas as pl
from jax.experimental.pallas import tpu as pltpu
from jax.experimental.pallas import tpu_sc as plsc

NC = 2
NS = 16
NW = NC * NS
CNTW = 16


def _sc_aggregate(N, D, E, x_neigh, src, dst, zero_sums, zero_cnt):
    e_per_w = E // NW
    ch = 80
    n_ch = e_per_w // ch
    rows_per_tile = N // NS

    mesh = plsc.VectorSubcoreMesh(core_axis_name="c", subcore_axis_name="s")

    @functools.partial(
        pl.kernel,
        out_type=[
            jax.ShapeDtypeStruct((NC, N, D), jnp.float32),
            jax.ShapeDtypeStruct((NC, N, CNTW), jnp.float32),
        ],
        mesh=mesh,
        scratch_types=[
            pltpu.VMEM((ch,), jnp.int32),
            pltpu.VMEM((ch,), jnp.int32),
            pltpu.VMEM((ch, D), jnp.float32),
            pltpu.VMEM((ch, CNTW), jnp.float32),
            pltpu.VMEM_SHARED((N, D), jnp.float32),
            pltpu.VMEM_SHARED((N, CNTW), jnp.float32),
            pltpu.SemaphoreType.DMA,
        ],
    )
    def agg(x_hbm, src_hbm, dst_hbm, zs_hbm, zc_hbm, out_sums, out_cnt,
            idx_s, idx_d, rows, ones, acc_sums, acc_cnt, sem):
        c = lax.axis_index("c")
        s = lax.axis_index("s")
        wid = c * NS + s

        r0 = s * rows_per_tile
        pltpu.sync_copy(zs_hbm.at[pl.ds(r0, rows_per_tile)],
                        acc_sums.at[pl.ds(r0, rows_per_tile)])
        pltpu.sync_copy(zc_hbm.at[pl.ds(r0, rows_per_tile)],
                        acc_cnt.at[pl.ds(r0, rows_per_tile)])

        def init_ones(i, _):
            ones[i] = jnp.ones((CNTW,), jnp.float32)
            return 0
        lax.fori_loop(0, ch, init_ones, 0)

        plsc.subcore_barrier()

        base = wid * e_per_w

        def body(i, _):
            off = base + i * ch
            pltpu.sync_copy(src_hbm.at[pl.ds(off, ch)], idx_s)
            pltpu.sync_copy(dst_hbm.at[pl.ds(off, ch)], idx_d)
            pltpu.async_copy(x_hbm.at[idx_s], rows, sem).wait()
            pltpu.sync_copy(rows, acc_sums.at[idx_d], add=True)
            pltpu.sync_copy(ones, acc_cnt.at[idx_d], add=True)
            return 0
        lax.fori_loop(0, n_ch, body, 0)

        plsc.subcore_barrier()

        pltpu.sync_copy(acc_sums.at[pl.ds(r0, rows_per_tile)],
                        out_sums.at[c, pl.ds(r0, rows_per_tile)])
        pltpu.sync_copy(acc_cnt.at[pl.ds(r0, rows_per_tile)],
                        out_cnt.at[c, pl.ds(r0, rows_per_tile)])

    return agg(x_neigh, src, dst, zero_sums, zero_cnt)


def _dense_body(x_ref, sums_ref, cnt_ref, wst_ref, wnt_ref, out_ref):
    ssum = sums_ref[0] + sums_ref[1]
    cnt = cnt_ref[0, :, 0:1] + cnt_ref[1, :, 0:1]
    h = ssum / jnp.maximum(cnt, 1.0)
    z = (jnp.dot(x_ref[...], wst_ref[...], precision=lax.Precision.HIGHEST,
                 preferred_element_type=jnp.float32)
         + jnp.dot(h, wnt_ref[...], precision=lax.Precision.HIGHEST,
                   preferred_element_type=jnp.float32))
    z = jnp.maximum(z, 0.0)
    nrm = jnp.sqrt(jnp.sum(z * z, axis=1, keepdims=True))
    out_ref[...] = z / jnp.where(nrm == 0.0, 1.0, nrm)


def _tc_dense(N, D, x_self, sums2, cnt2, WsT, WnT):
    bn = 1000
    grid = N // bn
    return pl.pallas_call(
        _dense_body,
        grid=(grid,),
        in_specs=[
            pl.BlockSpec((bn, D), lambda i: (i, 0)),
            pl.BlockSpec((NC, bn, D), lambda i: (0, i, 0)),
            pl.BlockSpec((NC, bn, CNTW), lambda i: (0, i, 0)),
            pl.BlockSpec((D, D), lambda i: (0, 0)),
            pl.BlockSpec((D, D), lambda i: (0, 0)),
        ],
        out_specs=pl.BlockSpec((bn, D), lambda i: (i, 0)),
        out_shape=jax.ShapeDtypeStruct((N, D), jnp.float32),
    )(x_self, sums2, cnt2, WsT, WnT)


def kernel(x_neigh, x_self, edge_index, W_self, W_neigh):
    N, D = x_neigh.shape
    E = edge_index.shape[1]
    src = edge_index[0]
    dst = edge_index[1]
    zero_sums = jnp.zeros((N, D), jnp.float32)
    zero_cnt = jnp.zeros((N, CNTW), jnp.float32)
    sums2, cnt2 = _sc_aggregate(N, D, E, x_neigh, src, dst, zero_sums, zero_cnt)
    return _tc_dense(N, D, x_self, sums2, cnt2, W_self.T, W_neigh.T)


# speedup vs baseline: 6.1419x; 6.1419x over previous
import functools

import jax
import jax.numpy as jnp
from jax import lax
from jax.experimental import pallas as pl
from jax.experimental.pallas import tpu as pltpu
from jax.experimental.pallas import tpu_sc as plsc

NC = 2
NS = 16
NW = NC * NS
L = 16


def _sc_aggregate(Np, D, E, x_neigh, src, dst):
    e_per_w = E // NW
    ch = 80
    n_ch = e_per_w // ch
    rows_per_tile = Np // NS
    n_stage = rows_per_tile // ch

    mesh = plsc.VectorSubcoreMesh(core_axis_name="c", subcore_axis_name="s")

    @functools.partial(
        pl.kernel,
        out_type=[
            jax.ShapeDtypeStruct((NC, Np, D), jnp.float32),
            jax.ShapeDtypeStruct((NW, Np), jnp.float32),
        ],
        mesh=mesh,
        compiler_params=pltpu.CompilerParams(needs_layout_passes=False),
        scratch_types=[
            pltpu.VMEM((ch,), jnp.int32),
            pltpu.VMEM((ch,), jnp.int32),
            pltpu.VMEM((ch, D), jnp.float32),
            pltpu.VMEM((Np,), jnp.float32),
            pltpu.VMEM_SHARED((Np, D), jnp.float32),
            pltpu.SemaphoreType.DMA,
        ],
    )
    def agg(x_hbm, src_hbm, dst_hbm, out_sums, out_cnt,
            idx_s, idx_d, rows, hist, acc_sums, sem):
        c = lax.axis_index("c")
        s = lax.axis_index("s")
        wid = c * NS + s
        r0 = s * rows_per_tile
        zeros16 = jnp.zeros((L,), jnp.float32)
        ones16 = jnp.ones((L,), jnp.float32)

        def z_rows(i, _):
            for j in range(D // L):
                rows[i, pl.ds(j * L, L)] = zeros16
            return 0
        lax.fori_loop(0, ch, z_rows, 0)

        def z_hist(i, _):
            hist[pl.ds(i * L, L)] = zeros16
            return 0
        lax.fori_loop(0, Np // L, z_hist, 0)

        def z_acc(k, _):
            pltpu.sync_copy(rows, acc_sums.at[pl.ds(r0 + k * ch, ch)])
            return 0
        lax.fori_loop(0, n_stage, z_acc, 0)

        plsc.subcore_barrier()

        base = wid * e_per_w

        def body(i, _):
            off = base + i * ch
            pltpu.sync_copy(src_hbm.at[pl.ds(off, ch)], idx_s)
            pltpu.sync_copy(dst_hbm.at[pl.ds(off, ch)], idx_d)
            pltpu.async_copy(x_hbm.at[idx_s], rows, sem).wait()
            pltpu.sync_copy(rows, acc_sums.at[idx_d], add=True)
            for j in range(ch // L):
                idx = idx_d[pl.ds(j * L, L)]
                plsc.addupdate_scatter(hist, [idx], ones16)
            return 0
        lax.fori_loop(0, n_ch, body, 0)

        plsc.subcore_barrier()

        def wout(k, _):
            rr = r0 + k * ch
            pltpu.sync_copy(acc_sums.at[pl.ds(rr, ch)], rows)
            pltpu.sync_copy(rows, out_sums.at[c, pl.ds(rr, ch)])
            return 0
        lax.fori_loop(0, n_stage, wout, 0)
        pltpu.sync_copy(hist, out_cnt.at[wid])

    return agg(x_neigh, src, dst)


def _dense_body(x_ref, sums_ref, cnt_ref, wst_ref, wnt_ref, out_ref):
    ssum = sums_ref[0] + sums_ref[1]
    cnt = lax.dot_general(cnt_ref[...], jnp.ones((NW, 1), jnp.float32),
                          (((0,), (0,)), ((), ())),
                          precision=lax.Precision.HIGHEST,
                          preferred_element_type=jnp.float32)
    h = ssum / jnp.maximum(cnt, 1.0)
    z = (jnp.dot(x_ref[...], wst_ref[...], precision=lax.Precision.HIGHEST,
                 preferred_element_type=jnp.float32)
         + jnp.dot(h, wnt_ref[...], precision=lax.Precision.HIGHEST,
                   preferred_element_type=jnp.float32))
    z = jnp.maximum(z, 0.0)
    nrm = jnp.sqrt(jnp.sum(z * z, axis=1, keepdims=True))
    out_ref[...] = z / jnp.where(nrm == 0.0, 1.0, nrm)


def _tc_dense(N, D, x_self, sums2, cnt2, WsT, WnT):
    bn = 1024
    grid = (N + bn - 1) // bn
    return pl.pallas_call(
        _dense_body,
        grid=(grid,),
        in_specs=[
            pl.BlockSpec((bn, D), lambda i: (i, 0)),
            pl.BlockSpec((NC, bn, D), lambda i: (0, i, 0)),
            pl.BlockSpec((NW, bn), lambda i: (0, i)),
            pl.BlockSpec((D, D), lambda i: (0, 0)),
            pl.BlockSpec((D, D), lambda i: (0, 0)),
        ],
        out_specs=pl.BlockSpec((bn, D), lambda i: (i, 0)),
        out_shape=jax.ShapeDtypeStruct((N, D), jnp.float32),
    )(x_self, sums2, cnt2, WsT, WnT)


def kernel(x_neigh, x_self, edge_index, W_self, W_neigh):
    N, D = x_neigh.shape
    E = edge_index.shape[1]
    src = edge_index[0]
    dst = edge_index[1]
    Np = ((N + 1280 - 1) // 1280) * 1280
    sums2, cnt2 = _sc_aggregate(Np, D, E, x_neigh, src, dst)
    return _tc_dense(N, D, x_self, sums2, cnt2, W_self.T, W_neigh.T)


# speedup vs baseline: 11.1511x; 1.8156x over previous
import functools

import jax
import jax.numpy as jnp
from jax import lax
from jax.experimental import pallas as pl
from jax.experimental.pallas import tpu as pltpu
from jax.experimental.pallas import tpu_sc as plsc

NC = 2
NS = 16
NW = NC * NS
L = 16
CH = 80


def _sc_aggregate(Np, D, E, x_neigh, src2, dst4):
    e_per_w = E // NW
    n_ch = e_per_w // CH
    n_pair = n_ch // 2
    tail = n_ch % 2
    rows_per_tile = Np // NS
    n_stage = rows_per_tile // CH

    mesh = plsc.VectorSubcoreMesh(core_axis_name="c", subcore_axis_name="s")

    @functools.partial(
        pl.kernel,
        out_type=[
            jax.ShapeDtypeStruct((NC, Np, D), jnp.float32),
            jax.ShapeDtypeStruct((NW, Np), jnp.float32),
        ],
        mesh=mesh,
        compiler_params=pltpu.CompilerParams(needs_layout_passes=False),
        scratch_types=[
            pltpu.VMEM((CH,), jnp.int32),
            pltpu.VMEM((CH,), jnp.int32),
            pltpu.VMEM((e_per_w,), jnp.int32),
            pltpu.VMEM((CH, D), jnp.float32),
            pltpu.VMEM((CH, D), jnp.float32),
            pltpu.VMEM((Np,), jnp.float32),
            pltpu.VMEM_SHARED((Np, D), jnp.float32),
            pltpu.SemaphoreType.DMA,
            pltpu.SemaphoreType.DMA,
            pltpu.SemaphoreType.DMA,
            pltpu.SemaphoreType.DMA,
            pltpu.SemaphoreType.DMA,
            pltpu.SemaphoreType.DMA,
        ],
    )
    def agg(x_hbm, src_hbm, dst_hbm, out_sums, out_cnt,
            i0, i1, idx_d, rows0, rows1, hist, acc_sums,
            semg0, semg1, sems0, sems1, semi0, semi1):
        c = lax.axis_index("c")
        s = lax.axis_index("s")
        wid = c * NS + s
        r0 = s * rows_per_tile
        zeros16 = jnp.zeros((L,), jnp.float32)
        ones16 = jnp.ones((L,), jnp.float32)

        base = wid * e_per_w
        pltpu.sync_copy(dst_hbm.at[pl.ds(base, e_per_w)], idx_d)

        def z_rows(i, _):
            for j in range(D // L):
                rows0[i, pl.ds(j * L, L)] = zeros16
            return 0
        lax.fori_loop(0, CH, z_rows, 0)

        def z_hist(i, _):
            hist[pl.ds(i * L, L)] = zeros16
            return 0
        lax.fori_loop(0, Np // L, z_hist, 0)

        def z_acc(k, _):
            pltpu.sync_copy(rows0, acc_sums.at[pl.ds(r0 + k * CH, CH)])
            return 0
        lax.fori_loop(0, n_stage, z_acc, 0)

        plsc.subcore_barrier()

        def do_hist(i):
            for j in range(CH // L):
                idx = idx_d[pl.ds(i * CH + j * L, L)]
                plsc.addupdate_scatter(hist, [idx], ones16)

        def pf_idx(i, ibuf, sem):
            return pltpu.async_copy(src_hbm.at[pl.ds(base + i * CH, CH)],
                                    ibuf, sem)

        def gather(ibuf, buf, sem):
            return pltpu.async_copy(x_hbm.at[ibuf], buf, sem)

        def scatter(i, buf, sem):
            return pltpu.async_copy(buf, acc_sums.at[idx_d.at[pl.ds(i * CH, CH)]],
                                    sem, add=True)

        last = n_ch - 1
        pf_idx(0, i0, semi0).wait()
        pf_idx(1, i1, semi1).wait()

        def pair(g, _):
            i = 2 * g
            dg0 = gather(i0, rows0, semg0)
            dg1 = gather(i1, rows1, semg1)
            dg0.wait()
            dp0 = pf_idx(jnp.minimum(i + 2, last), i0, semi0)
            ds0 = scatter(i, rows0, sems0)
            do_hist(i)
            dg1.wait()
            dp1 = pf_idx(jnp.minimum(i + 3, last), i1, semi1)
            ds1 = scatter(i + 1, rows1, sems1)
            do_hist(i + 1)
            ds0.wait()
            ds1.wait()
            dp0.wait()
            dp1.wait()
            return 0
        lax.fori_loop(0, n_pair, pair, 0)

        if tail:
            i = n_ch - 1
            dg0 = gather(i0, rows0, semg0)
            dg0.wait()
            ds0 = scatter(i, rows0, sems0)
            do_hist(i)
            ds0.wait()

        plsc.subcore_barrier()

        def wout(k, _):
            rr = r0 + k * CH
            pltpu.sync_copy(acc_sums.at[pl.ds(rr, CH)], rows0)
            pltpu.sync_copy(rows0, out_sums.at[c, pl.ds(rr, CH)])
            return 0
        lax.fori_loop(0, n_stage, wout, 0)
        pltpu.sync_copy(hist, out_cnt.at[wid])

    return agg(x_neigh, src2, dst4)


def _dense_body(x_ref, sums_ref, cnt_ref, wst_ref, wnt_ref, out_ref):
    ssum = sums_ref[0] + sums_ref[1]
    cnt = lax.dot_general(cnt_ref[...], jnp.ones((NW, 1), jnp.float32),
                          (((0,), (0,)), ((), ())),
                          preferred_element_type=jnp.float32)
    h = ssum / jnp.maximum(cnt, 1.0)
    z = (jnp.dot(x_ref[...], wst_ref[...],
                 preferred_element_type=jnp.float32)
         + jnp.dot(h, wnt_ref[...],
                   preferred_element_type=jnp.float32))
    z = jnp.maximum(z, 0.0)
    nrm = jnp.sqrt(jnp.sum(z * z, axis=1, keepdims=True))
    out_ref[...] = z / jnp.where(nrm == 0.0, 1.0, nrm)


def _tc_dense(N, D, x_self, sums2, cnt2, WsT, WnT):
    bn = 1024
    grid = (N + bn - 1) // bn
    return pl.pallas_call(
        _dense_body,
        grid=(grid,),
        in_specs=[
            pl.BlockSpec((bn, D), lambda i: (i, 0)),
            pl.BlockSpec((NC, bn, D), lambda i: (0, i, 0)),
            pl.BlockSpec((NW, bn), lambda i: (0, i)),
            pl.BlockSpec((D, D), lambda i: (0, 0)),
            pl.BlockSpec((D, D), lambda i: (0, 0)),
        ],
        out_specs=pl.BlockSpec((bn, D), lambda i: (i, 0)),
        out_shape=jax.ShapeDtypeStruct((N, D), jnp.float32),
    )(x_self, sums2, cnt2, WsT, WnT)


def kernel(x_neigh, x_self, edge_index, W_self, W_neigh):
    N, D = x_neigh.shape
    E = edge_index.shape[1]
    e_per_w = E // NW
    n_ch = e_per_w // CH
    src2 = edge_index[0]
    dst4 = edge_index[1]
    Np = ((N + 16 * CH - 1) // (16 * CH)) * (16 * CH)
    sums2, cnt2 = _sc_aggregate(Np, D, E, x_neigh, src2, dst4)
    return _tc_dense(N, D, x_self, sums2, cnt2, W_self.T, W_neigh.T)


# speedup vs baseline: 11.4604x; 1.0277x over previous
import functools

import jax
import jax.numpy as jnp
from jax import lax
from jax.experimental import pallas as pl
from jax.experimental.pallas import tpu as pltpu
from jax.experimental.pallas import tpu_sc as plsc

NC = 2
NS = 16
NW = NC * NS
L = 16
CH = 80


def _sc_aggregate(Np, D, E, x_neigh, src2, dst4):
    e_per_w = E // NW
    n_ch = e_per_w // CH
    n_pair = n_ch // 2
    tail = n_ch % 2
    rows_per_tile = Np // NS
    n_stage = rows_per_tile // CH

    mesh = plsc.VectorSubcoreMesh(core_axis_name="c", subcore_axis_name="s")

    @functools.partial(
        pl.kernel,
        out_type=[
            jax.ShapeDtypeStruct((NC, Np, D), jnp.float32),
            jax.ShapeDtypeStruct((NW, Np), jnp.float32),
        ],
        mesh=mesh,
        compiler_params=pltpu.CompilerParams(needs_layout_passes=False),
        scratch_types=[
            pltpu.VMEM((CH,), jnp.int32),
            pltpu.VMEM((CH,), jnp.int32),
            pltpu.VMEM((e_per_w,), jnp.int32),
            pltpu.VMEM((CH, D), jnp.float32),
            pltpu.VMEM((CH, D), jnp.float32),
            pltpu.VMEM((Np,), jnp.float32),
            pltpu.VMEM_SHARED((Np, D), jnp.float32),
            pltpu.SemaphoreType.DMA,
            pltpu.SemaphoreType.DMA,
            pltpu.SemaphoreType.DMA,
            pltpu.SemaphoreType.DMA,
            pltpu.SemaphoreType.DMA,
            pltpu.SemaphoreType.DMA,
        ],
    )
    def agg(x_hbm, src_hbm, dst_hbm, out_sums, out_cnt,
            i0, i1, idx_d, rows0, rows1, hist, acc_sums,
            semg0, semg1, sems0, sems1, semi0, semi1):
        c = lax.axis_index("c")
        s = lax.axis_index("s")
        wid = c * NS + s
        r0 = s * rows_per_tile
        zeros16 = jnp.zeros((L,), jnp.float32)
        ones16 = jnp.ones((L,), jnp.float32)

        base = wid * e_per_w
        pltpu.sync_copy(dst_hbm.at[pl.ds(base, e_per_w)], idx_d)

        def z_rows(i, _):
            for j in range(D // L):
                rows0[i, pl.ds(j * L, L)] = zeros16
            return 0
        lax.fori_loop(0, CH, z_rows, 0)

        def z_hist(i, _):
            hist[pl.ds(i * L, L)] = zeros16
            return 0
        lax.fori_loop(0, Np // L, z_hist, 0)

        for k in range(n_stage):
            pltpu.async_copy(rows0, acc_sums.at[pl.ds(r0 + k * CH, CH)], semg0)
        for k in range(n_stage):
            pltpu.make_async_copy(rows0, acc_sums.at[pl.ds(r0 + k * CH, CH)],
                                  semg0).wait()

        plsc.subcore_barrier()

        def do_hist(i):
            for j in range(CH // L):
                idx = idx_d[pl.ds(i * CH + j * L, L)]
                plsc.addupdate_scatter(hist, [idx], ones16)

        def pf_idx(i, ibuf, sem):
            return pltpu.async_copy(src_hbm.at[pl.ds(base + i * CH, CH)],
                                    ibuf, sem)

        def gather(ibuf, buf, sem):
            return pltpu.async_copy(x_hbm.at[ibuf], buf, sem)

        def scatter(i, buf, sem):
            return pltpu.async_copy(buf, acc_sums.at[idx_d.at[pl.ds(i * CH, CH)]],
                                    sem, add=True)

        last = n_ch - 1

        def wait_gather(ibuf, buf, sem):
            pltpu.make_async_copy(x_hbm.at[ibuf], buf, sem).wait()

        pf_idx(0, i0, semi0).wait()
        pf_idx(1, i1, semi1).wait()
        gather(i0, rows0, semg0)
        gather(i1, rows1, semg1)

        def pair(g, _):
            i = 2 * g
            wait_gather(i0, rows0, semg0)
            dp0 = pf_idx(jnp.minimum(i + 2, last), i0, semi0)
            ds0 = scatter(i, rows0, sems0)
            do_hist(i)
            wait_gather(i1, rows1, semg1)
            dp1 = pf_idx(jnp.minimum(i + 3, last), i1, semi1)
            ds1 = scatter(i + 1, rows1, sems1)
            do_hist(i + 1)
            ds0.wait()
            dp0.wait()
            gather(i0, rows0, semg0)
            ds1.wait()
            dp1.wait()
            gather(i1, rows1, semg1)
            return 0
        lax.fori_loop(0, n_pair, pair, 0)

        if tail:
            i = n_ch - 1
            wait_gather(i0, rows0, semg0)
            ds0 = scatter(i, rows0, sems0)
            do_hist(i)
            ds0.wait()
            wait_gather(i1, rows1, semg1)

        plsc.subcore_barrier()

        def wout(k, _):
            rr = r0 + k * CH
            pltpu.sync_copy(acc_sums.at[pl.ds(rr, CH)], rows0)
            pltpu.sync_copy(rows0, out_sums.at[c, pl.ds(rr, CH)])
            return 0
        lax.fori_loop(0, n_stage, wout, 0)
        pltpu.sync_copy(hist, out_cnt.at[wid])

    return agg(x_neigh, src2, dst4)


def _dense_body(x_ref, sums_ref, cnt_ref, wst_ref, wnt_ref, out_ref):
    ssum = sums_ref[0] + sums_ref[1]
    cnt = lax.dot_general(cnt_ref[...], jnp.ones((NW, 1), jnp.float32),
                          (((0,), (0,)), ((), ())),
                          preferred_element_type=jnp.float32)
    h = ssum / jnp.maximum(cnt, 1.0)
    z = (jnp.dot(x_ref[...], wst_ref[...],
                 preferred_element_type=jnp.float32)
         + jnp.dot(h, wnt_ref[...],
                   preferred_element_type=jnp.float32))
    z = jnp.maximum(z, 0.0)
    nrm = jnp.sqrt(jnp.sum(z * z, axis=1, keepdims=True))
    out_ref[...] = z / jnp.where(nrm == 0.0, 1.0, nrm)


def _tc_dense(N, D, x_self, sums2, cnt2, WsT, WnT):
    bn = 1024
    grid = (N + bn - 1) // bn
    return pl.pallas_call(
        _dense_body,
        grid=(grid,),
        in_specs=[
            pl.BlockSpec((bn, D), lambda i: (i, 0)),
            pl.BlockSpec((NC, bn, D), lambda i: (0, i, 0)),
            pl.BlockSpec((NW, bn), lambda i: (0, i)),
            pl.BlockSpec((D, D), lambda i: (0, 0)),
            pl.BlockSpec((D, D), lambda i: (0, 0)),
        ],
        out_specs=pl.BlockSpec((bn, D), lambda i: (i, 0)),
        out_shape=jax.ShapeDtypeStruct((N, D), jnp.float32),
    )(x_self, sums2, cnt2, WsT, WnT)


def kernel(x_neigh, x_self, edge_index, W_self, W_neigh):
    N, D = x_neigh.shape
    E = edge_index.shape[1]
    e_per_w = E // NW
    n_ch = e_per_w // CH
    src2 = edge_index[0]
    dst4 = edge_index[1]
    Np = ((N + 16 * CH - 1) // (16 * CH)) * (16 * CH)
    sums2, cnt2 = _sc_aggregate(Np, D, E, x_neigh, src2, dst4)
    return _tc_dense(N, D, x_self, sums2, cnt2, W_self.T, W_neigh.T)
